# Initial kernel scaffold; baseline (speedup 1.0000x reference)
import jax
import jax.numpy as jnp
from jax.experimental import pallas as pl


def kernel(idx, table):
    raise NotImplementedError("write your pallas kernel here")


# speedup vs baseline: 1.3422x; 1.3422x over previous
import functools

import jax
import jax.numpy as jnp
from jax import lax
from jax.experimental import pallas as pl
from jax.experimental.pallas import tpu as pltpu
from jax.experimental.pallas import tpu_sc as plsc

EMBED_DIM = 512
NUM_IDX = 256 * 1025
NC = 2
NS = 16
NW = NC * NS
BPW = NUM_IDX // NW
CHUNK = 112
NCHUNK = 74
TAIL = BPW - (NCHUNK - 1) * CHUNK

_mesh = plsc.VectorSubcoreMesh(
    core_axis_name="c", subcore_axis_name="s", num_cores=NC, num_subcores=NS
)


@functools.partial(
    pl.kernel,
    out_type=jax.ShapeDtypeStruct((NUM_IDX, EMBED_DIM), jnp.float32),
    mesh=_mesh,
    scratch_types=[
        pltpu.VMEM((NCHUNK, CHUNK), jnp.int32),
        pltpu.VMEM((2, CHUNK, EMBED_DIM), jnp.float32),
        pltpu.SemaphoreType.DMA,
        pltpu.SemaphoreType.DMA,
        pltpu.SemaphoreType.DMA,
        pltpu.SemaphoreType.DMA,
    ],
)
def _gather_kernel(idx_hbm, table_hbm, out_hbm, idx_v, rows_v, g0, g1, o0, o1):
    wid = lax.axis_index("s") * NC + lax.axis_index("c")
    base = wid * BPW
    gsem = (g0, g1)
    osem = (o0, o1)

    pltpu.sync_copy(idx_hbm.at[wid], idx_v)

    def gather_start(k, b):
        pltpu.make_async_copy(
            table_hbm.at[idx_v.at[k]], rows_v.at[b], gsem[b]
        ).start()

    def gather_wait(b):
        pltpu.make_async_copy(
            table_hbm.at[pl.ds(0, CHUNK)], rows_v.at[b], gsem[b]
        ).wait()

    def out_start(k, b):
        pltpu.make_async_copy(
            rows_v.at[b], out_hbm.at[pl.ds(base + k * CHUNK, CHUNK)], osem[b]
        ).start()

    def out_wait(b):
        pltpu.make_async_copy(
            table_hbm.at[pl.ds(0, CHUNK)], rows_v.at[b], osem[b]
        ).wait()

    gather_start(0, 0)
    gather_start(1, 1)

    def body(kk, carry):
        k0 = kk * 2
        for b in (0, 1):
            gather_wait(b)
            out_start(k0 + b, b)
        for b in (0, 1):

            @pl.when(k0 + b + 2 < NCHUNK)
            def _():
                out_wait(b)
                gather_start(k0 + b + 2, b)

        return carry

    lax.fori_loop(0, NCHUNK // 2 - 1, body, 0)

    gather_wait(0)
    out_start(NCHUNK - 2, 0)
    gather_wait(1)
    pltpu.make_async_copy(
        rows_v.at[1, pl.ds(0, TAIL)],
        out_hbm.at[pl.ds(base + (NCHUNK - 1) * CHUNK, TAIL)],
        osem[1],
    ).start()
    out_wait(0)
    pltpu.make_async_copy(
        table_hbm.at[pl.ds(0, TAIL)], rows_v.at[1, pl.ds(0, TAIL)], osem[1]
    ).wait()


def kernel(idx, table):
    idx_flat = idx.reshape(NW, BPW).astype(jnp.int32)
    idx_pad = jnp.pad(idx_flat, ((0, 0), (0, NCHUNK * CHUNK - BPW)))
    idx_r = idx_pad.reshape(NW, NCHUNK, CHUNK)
    out = _gather_kernel(idx_r, table)
    return out.reshape(idx.shape[0], idx.shape[1], EMBED_DIM)


# speedup vs baseline: 2.0625x; 1.5367x over previous
import functools

import jax
import jax.numpy as jnp
from jax import lax
from jax.experimental import pallas as pl
from jax.experimental.pallas import tpu as pltpu
from jax.experimental.pallas import tpu_sc as plsc

EMBED_DIM = 512
NIMG = 256
NPOS = 1025
NC = 2
NS = 16
NW = NC * NS
IPW = NIMG // NW
CHUNK = 64
CPI = 1024 // CHUNK
NCHUNK = IPW * CPI
NBUF = 3

_mesh = plsc.VectorSubcoreMesh(
    core_axis_name="c", subcore_axis_name="s", num_cores=NC, num_subcores=NS
)


@functools.partial(
    pl.kernel,
    out_type=(
        jax.ShapeDtypeStruct((NIMG, NPOS, EMBED_DIM), jnp.float32),
        jax.ShapeDtypeStruct((NIMG, EMBED_DIM), jnp.float32),
    ),
    mesh=_mesh,
    scratch_types=[
        pltpu.VMEM((NCHUNK, CHUNK), jnp.int32),
        pltpu.VMEM((NBUF, CHUNK, EMBED_DIM), jnp.float32),
        pltpu.VMEM((IPW,), jnp.int32),
        pltpu.VMEM((IPW, EMBED_DIM), jnp.float32),
        pltpu.SemaphoreType.DMA,
        pltpu.SemaphoreType.DMA,
        pltpu.SemaphoreType.DMA,
        pltpu.SemaphoreType.DMA,
        pltpu.SemaphoreType.DMA,
        pltpu.SemaphoreType.DMA,
        pltpu.SemaphoreType.DMA,
    ],
)
def _sc_gather(idx_hbm, tidx_hbm, table_hbm, out_hbm, tails_hbm,
               idx_v, rows_v, tidx_v, trows_v,
               g0, g1, g2, o0, o1, o2, tsem):
    wid = lax.axis_index("s") * NC + lax.axis_index("c")
    img0 = wid * IPW
    gsem = (g0, g1, g2)
    osem = (o0, o1, o2)

    pltpu.sync_copy(idx_hbm.at[wid], idx_v)
    pltpu.sync_copy(tidx_hbm.at[wid], tidx_v)

    pltpu.make_async_copy(table_hbm.at[tidx_v], trows_v, tsem).start()

    def gather_start(k, b):
        pltpu.make_async_copy(
            table_hbm.at[idx_v.at[k]], rows_v.at[b], gsem[b]
        ).start()

    def gather_wait(b):
        pltpu.make_async_copy(
            table_hbm.at[pl.ds(0, CHUNK)], rows_v.at[b], gsem[b]
        ).wait()

    def out_start(k, b):
        img = img0 + k // CPI
        r0 = (k % CPI) * CHUNK
        pltpu.make_async_copy(
            rows_v.at[b], out_hbm.at[img, pl.ds(r0, CHUNK)], osem[b]
        ).start()

    def out_wait(b):
        pltpu.make_async_copy(
            table_hbm.at[pl.ds(0, CHUNK)], rows_v.at[b], osem[b]
        ).wait()

    for b in range(NBUF):
        gather_start(b, b)

    def body(kk, carry):
        k0 = kk * NBUF
        for b in range(NBUF):
            gather_wait(b)
            out_start(k0 + b, b)
        for b in range(NBUF):

            @pl.when(k0 + b + NBUF < NCHUNK)
            def _():
                out_wait(b)
                gather_start(k0 + b + NBUF, b)

        return carry

    lax.fori_loop(0, NCHUNK // NBUF, body, 0)
    rem = NCHUNK - (NCHUNK // NBUF) * NBUF
    for b in range(rem):
        gather_wait(b)
        out_start(NCHUNK - rem + b, b)
    for b in range(NBUF):
        out_wait(b)

    pltpu.make_async_copy(
        table_hbm.at[pl.ds(0, IPW)], trows_v, tsem
    ).wait()
    pltpu.sync_copy(trows_v, tails_hbm.at[pl.ds(img0, IPW)])


def _merge_body(big_ref, tails_ref, out_ref):
    out_ref[...] = jnp.broadcast_to(
        tails_ref[...][:, None, :], (NIMG, 8, EMBED_DIM)
    )


_merge = pl.pallas_call(
    _merge_body,
    out_shape=jax.ShapeDtypeStruct((NIMG, NPOS, EMBED_DIM), jnp.float32),
    grid=(1,),
    in_specs=[
        pl.BlockSpec(memory_space=pl.ANY),
        pl.BlockSpec((NIMG, EMBED_DIM), lambda i: (0, 0)),
    ],
    out_specs=pl.BlockSpec((NIMG, 8, EMBED_DIM), lambda i: (0, 1024 // 8, 0)),
    input_output_aliases={0: 0},
)


def kernel(idx, table):
    idx_i32 = idx.astype(jnp.int32)
    idx_main = idx_i32[:, :1024].reshape(NW, NCHUNK, CHUNK)
    idx_tail = idx_i32[:, 1024].reshape(NW, IPW)
    big, tails = _sc_gather(idx_main, idx_tail, table)
    return _merge(big, tails)
